# baseline (speedup 1.0000x reference)
import jax
import jax.numpy as jnp
from jax.experimental import pallas as pl
from jax.experimental.pallas import tpu as pltpu

_B, _S, _H = 1, 2048, 768
_E = 8
_FF = _H * 4
_FFB = 1024
_NFF = _FF // _FFB
_NBLK = _E * _NFF


def _moe_body(x_ref, gw_ref, gb_ref, w1_ref, b1_ref, w2_ref, b2_ref,
              out_ref, probs_ref, xbf_ref, pb_ref, g_ref):
    n = pl.program_id(0)

    @pl.when(n == 0)
    def _init():
        logits = jnp.dot(x_ref[...], gw_ref[...],
                         preferred_element_type=jnp.float32) + gb_ref[0]
        m = jnp.max(logits, axis=-1, keepdims=True)
        ex = jnp.exp(logits - m)
        p = ex / jnp.sum(ex, axis=-1, keepdims=True)
        p = p / jnp.sum(p, axis=-1, keepdims=True)
        probs_ref[...] = p
        xbf_ref[...] = x_ref[...].astype(jnp.bfloat16)
        out_ref[...] = jnp.dot(p, b2_ref[...],
                               preferred_element_type=jnp.float32)

    @pl.when((n >= 1) & ((n - 1) % _NFF == 0))
    def _per_expert():
        e_prev = (n - 1) // _NFF
        lane = jax.lax.broadcasted_iota(jnp.int32, (_S, _E), 1)
        pcol = jnp.sum(jnp.where(lane == e_prev, probs_ref[...], 0.0),
                       axis=1, keepdims=True)
        pb_ref[...] = jnp.broadcast_to(pcol, (_S, _H))

    @pl.when(n >= 1)
    def _second():
        d2 = jnp.dot(g_ref[...], w2_ref[0].astype(jnp.bfloat16),
                     preferred_element_type=jnp.float32)
        out_ref[...] += pb_ref[...] * d2

    @pl.when(n < _NBLK)
    def _first():
        h = jnp.dot(xbf_ref[...], w1_ref[0].astype(jnp.bfloat16),
                    preferred_element_type=jnp.float32)
        h = h + b1_ref[0, 0]
        h = 0.5 * h * (1.0 + jax.lax.erf(h * 0.7071067811865476))
        g_ref[...] = h.astype(jnp.bfloat16)


def _w1_idx(n):
    nc = jnp.minimum(n, _NBLK - 1)
    return (nc // _NFF, 0, nc % _NFF)


def _b1_idx(n):
    nc = jnp.minimum(n, _NBLK - 1)
    return (nc // _NFF, 0, nc % _NFF)


def _w2_idx(n):
    npv = jnp.maximum(n - 1, 0)
    return (npv // _NFF, npv % _NFF, 0)


@jax.jit
def kernel(x, gate_w, gate_b, w1, b1, w2, b2):
    b, s, h = x.shape
    xf = x.reshape(-1, h)
    t = xf.shape[0]

    out = pl.pallas_call(
        _moe_body,
        grid=(_NBLK + 1,),
        in_specs=[
            pl.BlockSpec((t, h), lambda n: (0, 0)),
            pl.BlockSpec((h, _E), lambda n: (0, 0)),
            pl.BlockSpec((1, _E), lambda n: (0, 0)),
            pl.BlockSpec((1, h, _FFB), _w1_idx),
            pl.BlockSpec((1, 1, _FFB), _b1_idx),
            pl.BlockSpec((1, _FFB, h), _w2_idx),
            pl.BlockSpec((_E, h), lambda n: (0, 0)),
        ],
        out_specs=pl.BlockSpec((t, h), lambda n: (0, 0)),
        out_shape=jax.ShapeDtypeStruct((t, h), jnp.float32),
        scratch_shapes=[pltpu.VMEM((t, _E), jnp.float32),
                        pltpu.VMEM((t, h), jnp.bfloat16),
                        pltpu.VMEM((t, h), jnp.float32),
                        pltpu.VMEM((t, _FFB), jnp.bfloat16)],
        compiler_params=pltpu.CompilerParams(
            dimension_semantics=("arbitrary",),
        ),
    )(xf, gate_w, gate_b.reshape(1, _E), w1, b1.reshape(_E, 1, _FF), w2, b2)
    return out.reshape(b, s, h)


# speedup vs baseline: 1.1230x; 1.1230x over previous
import jax
import jax.numpy as jnp
from jax.experimental import pallas as pl
from jax.experimental.pallas import tpu as pltpu

_B, _S, _H = 1, 2048, 768
_E = 8
_FF = _H * 4
_FFB = 1024
_NFF = _FF // _FFB


def _moe_body(x_ref, gw_ref, gb_ref, w1_ref, b1_ref, w2_ref, b2_ref,
              out_ref, probs_ref, pb_ref):
    e = pl.program_id(0)
    f = pl.program_id(1)

    @pl.when((e == 0) & (f == 0))
    def _init():
        logits = jnp.dot(x_ref[...], gw_ref[...],
                         preferred_element_type=jnp.float32) + gb_ref[0]
        m = jnp.max(logits, axis=-1, keepdims=True)
        ex = jnp.exp(logits - m)
        p = ex / jnp.sum(ex, axis=-1, keepdims=True)
        p = p / jnp.sum(p, axis=-1, keepdims=True)
        probs_ref[...] = p
        out_ref[...] = jnp.dot(p, b2_ref[...],
                               preferred_element_type=jnp.float32)

    @pl.when(f == 0)
    def _per_expert():
        lane = jax.lax.broadcasted_iota(jnp.int32, (_S, _E), 1)
        pcol = jnp.sum(jnp.where(lane == e, probs_ref[...], 0.0),
                       axis=1, keepdims=True)
        pb_ref[...] = jnp.broadcast_to(pcol, (_S, _H))

    h = jnp.dot(x_ref[...], w1_ref[0], preferred_element_type=jnp.float32)
    h = h + b1_ref[0, 0]
    h = 0.5 * h * (1.0 + jax.lax.erf(h * 0.7071067811865476))
    d2 = jnp.dot(h, w2_ref[0], preferred_element_type=jnp.float32)
    out_ref[...] += pb_ref[...] * d2


@jax.jit
def kernel(x, gate_w, gate_b, w1, b1, w2, b2):
    b, s, h = x.shape
    xf = x.reshape(-1, h)
    t = xf.shape[0]

    out = pl.pallas_call(
        _moe_body,
        grid=(_E, _NFF),
        in_specs=[
            pl.BlockSpec((t, h), lambda e, f: (0, 0)),
            pl.BlockSpec((h, _E), lambda e, f: (0, 0)),
            pl.BlockSpec((1, _E), lambda e, f: (0, 0)),
            pl.BlockSpec((1, h, _FFB), lambda e, f: (e, 0, f)),
            pl.BlockSpec((1, 1, _FFB), lambda e, f: (e, 0, f)),
            pl.BlockSpec((1, _FFB, h), lambda e, f: (e, f, 0)),
            pl.BlockSpec((_E, h), lambda e, f: (0, 0)),
        ],
        out_specs=pl.BlockSpec((t, h), lambda e, f: (0, 0)),
        out_shape=jax.ShapeDtypeStruct((t, h), jnp.float32),
        scratch_shapes=[pltpu.VMEM((t, _E), jnp.float32),
                        pltpu.VMEM((t, h), jnp.float32)],
        compiler_params=pltpu.CompilerParams(
            dimension_semantics=("arbitrary", "arbitrary"),
        ),
    )(xf, gate_w, gate_b.reshape(1, _E), w1, b1.reshape(_E, 1, _FF), w2, b2)
    return out.reshape(b, s, h)
